# Initial kernel scaffold; baseline (speedup 1.0000x reference)
import functools

import jax
import jax.numpy as jnp
from jax import lax
from jax.experimental import pallas as pl
from jax.experimental.pallas import tpu as pltpu
from jax.experimental.pallas import tpu_sc as plsc

N_NODES = 10000
N_PAD = 10240
N_EDGES = 320000
D = 128

N_TILES = 32
EPT = N_EDGES // N_TILES
CH = 80
NCH = EPT // CH
ROWS_PT = N_PAD // 16


def _prologue_body(h_ref, wfc_ref, wat2_ref, z_ref, st_ref):
    hb = h_ref[...]
    z = lax.dot_general(
        hb, wfc_ref[...],
        dimension_numbers=(((1,), (1,)), ((), ())),
        preferred_element_type=jnp.float32,
        precision=lax.Precision.HIGHEST,
    )
    z_ref[...] = z
    st = lax.dot_general(
        wat2_ref[...], z,
        dimension_numbers=(((0,), (1,)), ((), ())),
        preferred_element_type=jnp.float32,
        precision=lax.Precision.HIGHEST,
    )
    st_ref[...] = st


def _prologue(h_pad, w_fc, w_at2):
    nblk = N_PAD // 1024
    return pl.pallas_call(
        _prologue_body,
        grid=(nblk,),
        in_specs=[
            pl.BlockSpec((1024, D), lambda i: (i, 0)),
            pl.BlockSpec((D, D), lambda i: (0, 0)),
            pl.BlockSpec((D, 2), lambda i: (0, 0)),
        ],
        out_specs=[
            pl.BlockSpec((1024, D), lambda i: (i, 0)),
            pl.BlockSpec((2, 1024), lambda i: (0, i)),
        ],
        out_shape=[
            jax.ShapeDtypeStruct((N_PAD, D), jnp.float32),
            jax.ShapeDtypeStruct((2, N_PAD), jnp.float32),
        ],
    )(h_pad, w_fc, w_at2)


_SC_MESH = plsc.VectorSubcoreMesh(core_axis_name="c", subcore_axis_name="s")


@functools.partial(
    pl.kernel,
    out_type=[
        jax.ShapeDtypeStruct((2 * N_PAD,), jnp.float32),
        jax.ShapeDtypeStruct((2, N_PAD, D), jnp.float32),
    ],
    mesh=_SC_MESH,
    scratch_types=[
        pltpu.VMEM((N_NODES,), jnp.float32),
        pltpu.VMEM((N_NODES,), jnp.float32),
        pltpu.VMEM((EPT,), jnp.int32),
        pltpu.VMEM((EPT,), jnp.int32),
        pltpu.VMEM((NCH, CH), jnp.int32),
        pltpu.VMEM((NCH, CH), jnp.float32),
        pltpu.VMEM((CH, D), jnp.float32),
        pltpu.VMEM((CH, D), jnp.float32),
        pltpu.VMEM((ROWS_PT,), jnp.float32),
        pltpu.VMEM_SHARED((N_PAD, D), jnp.float32),
        pltpu.VMEM_SHARED((N_PAD,), jnp.float32),
    ],
)
def _sc_main(z_hbm, s_hbm, t_hbm, src_hbm, dst_hbm,
             den_out, hout_out,
             s_v, t_v, src_v, dst_v, dst2_v, ee_v, rows_v, zero_v, zden_v,
             hout_sh, den_sh):
    cid = lax.axis_index("c")
    sid = lax.axis_index("s")
    wid = cid * 16 + sid
    ebase = wid * EPT

    pltpu.sync_copy(s_hbm.at[pl.ds(0, N_NODES)], s_v)
    pltpu.sync_copy(t_hbm.at[pl.ds(0, N_NODES)], t_v)
    pltpu.sync_copy(src_hbm.at[pl.ds(ebase, EPT)], src_v)
    pltpu.sync_copy(dst_hbm.at[pl.ds(ebase, EPT)], dst_v)

    z16 = jnp.zeros((16,), jnp.float32)

    @pl.loop(0, CH)
    def _zrow(r):
        for q in range(D // 16):
            zero_v[r, pl.ds(q * 16, 16)] = z16

    @pl.loop(0, ROWS_PT // 16)
    def _zden(k):
        zden_v[pl.ds(k * 16, 16)] = z16

    rowbase = sid * ROWS_PT
    for q in range(ROWS_PT // CH):
        pltpu.sync_copy(zero_v, hout_sh.at[pl.ds(rowbase + q * CH, CH)])
    pltpu.sync_copy(zden_v, den_sh.at[pl.ds(rowbase, ROWS_PT)])
    plsc.subcore_barrier()

    @pl.loop(0, NCH)
    def _logits(j):
        for g in range(CH // 16):
            off = j * CH + g * 16
            src16 = src_v[pl.ds(off, 16)]
            dst16 = dst_v[pl.ds(off, 16)]
            sg = plsc.load_gather(s_v, [src16])
            tg = plsc.load_gather(t_v, [dst16])
            a = sg + tg
            e = jnp.maximum(a, a * 0.01)
            ee_v[j, pl.ds(g * 16, 16)] = jnp.exp(e)
            dst2_v[j, pl.ds(g * 16, 16)] = dst16

    @pl.loop(0, NCH)
    def _chunk(j):
        pltpu.sync_copy(z_hbm.at[src_v.at[pl.ds(j * CH, CH)]], rows_v)

        @pl.loop(0, CH)
        def _scale(i):
            w = ee_v[j, i]
            for q in range(D // 16):
                sl = pl.ds(q * 16, 16)
                rows_v[i, sl] = rows_v[i, sl] * w

        pltpu.sync_copy(ee_v.at[j], den_sh.at[dst2_v.at[j]], add=True)
        pltpu.sync_copy(rows_v, hout_sh.at[dst2_v.at[j]], add=True)

    plsc.subcore_barrier()

    pltpu.sync_copy(den_sh.at[pl.ds(rowbase, ROWS_PT)],
                    den_out.at[pl.ds(cid * N_PAD + rowbase, ROWS_PT)])
    for q in range(ROWS_PT // CH):
        rb = rowbase + q * CH
        pltpu.sync_copy(hout_sh.at[pl.ds(rb, CH)], hout_out.at[cid, pl.ds(rb, CH)])


def _epilogue_body(hp_ref, dp_ref, out_ref):
    d = dp_ref[0] + dp_ref[1]
    dsafe = jnp.where(d == 0.0, 1.0, d)
    hsum = hp_ref[0] + hp_ref[1]
    out_ref[...] = hsum * (1.0 / dsafe)


def _epilogue(hout_part, dcol):
    nblk = N_PAD // 1024
    return pl.pallas_call(
        _epilogue_body,
        grid=(nblk,),
        in_specs=[
            pl.BlockSpec((2, 1024, D), lambda i: (0, i, 0)),
            pl.BlockSpec((2, 1024, 1), lambda i: (0, i, 0)),
        ],
        out_specs=pl.BlockSpec((1024, D), lambda i: (i, 0)),
        out_shape=jax.ShapeDtypeStruct((N_PAD, D), jnp.float32),
    )(hout_part, dcol)


def kernel(h, edge_index, W_fc, W_attn):
    src = edge_index[0].astype(jnp.int32)
    dst = edge_index[1].astype(jnp.int32)
    h_pad = jnp.pad(h, ((0, N_PAD - N_NODES), (0, 0)))
    w_at2 = jnp.stack([W_attn[0, :D], W_attn[0, D:]], axis=1)

    z, st = _prologue(h_pad, W_fc, w_at2)
    s = st[0]
    t = st[1]

    den_flat, hout_part = _sc_main(z, s, t, src, dst)
    dcol = den_flat.reshape(2, N_PAD, 1)

    out = _epilogue(hout_part, dcol)
    return out[:N_NODES]


# speedup vs baseline: 15.1844x; 15.1844x over previous
import dataclasses
import functools

import jax
import jax.numpy as jnp
from jax import lax
from jax.experimental import pallas as pl
from jax.experimental.pallas import tpu as pltpu
from jax.experimental.pallas import tpu_sc as plsc

from jax._src.config import enable_x64 as _enable_x64

N_NODES = 10000
N_PAD = 10240
N_EDGES = 320000
D = 128
DH = D // 2

EPS = N_EDGES // 16
BLK = 800
NBLK = EPS // BLK
CH = 80
NCH = EPS // CH
CPB = BLK // CH
ROWS_PT = N_PAD // 16


def _i32(x):
    return jnp.int32(x)


def _prologue_body(h_ref, wfc_ref, wat2_ref, z_ref, st_ref):
    hb = h_ref[...]
    z = lax.dot_general(
        hb, wfc_ref[...],
        dimension_numbers=(((1,), (1,)), ((), ())),
        preferred_element_type=jnp.float32,
        precision=lax.Precision.HIGHEST,
    )
    z_ref[...] = z
    st = lax.dot_general(
        wat2_ref[...], z,
        dimension_numbers=(((0,), (1,)), ((), ())),
        preferred_element_type=jnp.float32,
        precision=lax.Precision.HIGHEST,
    )
    st_ref[...] = st


def _prologue(h_pad, w_fc, w_at2):
    nblk = N_PAD // 1024
    return pl.pallas_call(
        _prologue_body,
        grid=(nblk,),
        in_specs=[
            pl.BlockSpec((1024, D), lambda i: (i, 0)),
            pl.BlockSpec((D, D), lambda i: (0, 0)),
            pl.BlockSpec((D, 2), lambda i: (0, 0)),
        ],
        out_specs=[
            pl.BlockSpec((1024, D), lambda i: (i, 0)),
            pl.BlockSpec((2, 1024), lambda i: (0, i)),
        ],
        out_shape=[
            jax.ShapeDtypeStruct((N_PAD, D), jnp.float32),
            jax.ShapeDtypeStruct((2, N_PAD), jnp.float32),
        ],
    )(h_pad, w_fc, w_at2)


_SC_MESH = plsc.VectorSubcoreMesh(core_axis_name="c", subcore_axis_name="s")

_SC_CP = pltpu.CompilerParams()
_sc_fields = pltpu.CompilerParams.__dataclass_fields__
if "needs_layout_passes" in _sc_fields:
    _SC_CP = dataclasses.replace(_SC_CP, needs_layout_passes=False)
if "use_tc_tiling_on_sc" in _sc_fields:
    _SC_CP = dataclasses.replace(_SC_CP, use_tc_tiling_on_sc=False)


@functools.partial(
    pl.kernel,
    out_type=[
        jax.ShapeDtypeStruct((2 * N_PAD,), jnp.float32),
        jax.ShapeDtypeStruct((2, N_PAD, DH), jnp.float32),
    ],
    mesh=_SC_MESH,
    compiler_params=_SC_CP,
    scratch_types=[
        pltpu.VMEM((N_NODES,), jnp.float32),
        pltpu.VMEM((N_NODES,), jnp.float32),
        pltpu.VMEM((BLK,), jnp.int32),
        pltpu.VMEM((BLK,), jnp.int32),
        pltpu.VMEM((NCH, CH), jnp.int32),
        pltpu.VMEM((NCH, CH), jnp.int32),
        pltpu.VMEM((NCH, CH), jnp.float32),
        pltpu.VMEM((CH, DH), jnp.float32),
        pltpu.VMEM((ROWS_PT,), jnp.float32),
        pltpu.VMEM_SHARED((N_PAD, DH), jnp.float32),
        pltpu.VMEM_SHARED((N_PAD,), jnp.float32),
    ],
)
def _sc_main(zflat_hbm, s_hbm, t_hbm, src_hbm, dst_hbm,
             den_out, hout_out,
             s_v, t_v, src_blk, dst_blk, idx2_v, dst2_v, ee_v,
             rows_v, zden_v,
             hout_sh, den_sh):
    cid = lax.axis_index("c").astype(jnp.int32)
    sid = lax.axis_index("s").astype(jnp.int32)
    ebase = sid * _i32(EPS)

    pltpu.sync_copy(s_hbm.at[pl.ds(0, N_NODES)], s_v)
    pltpu.sync_copy(t_hbm.at[pl.ds(0, N_NODES)], t_v)

    z16 = jnp.zeros((16,), jnp.float32)

    @pl.loop(0, CH)
    def _zrow(r):
        for q in range(DH // 16):
            rows_v[r, pl.ds(q * 16, 16)] = z16

    @pl.loop(0, ROWS_PT // 16)
    def _zden(k):
        zden_v[pl.ds(k * _i32(16), 16)] = z16

    rowbase = sid * _i32(ROWS_PT)
    for q in range(ROWS_PT // CH):
        pltpu.sync_copy(rows_v, hout_sh.at[pl.ds(rowbase + _i32(q * CH), CH)])
    pltpu.sync_copy(zden_v, den_sh.at[pl.ds(rowbase, ROWS_PT)])
    plsc.subcore_barrier()

    @pl.loop(0, NBLK)
    def _blk(b):
        boff = ebase + b * _i32(BLK)
        pltpu.sync_copy(src_hbm.at[pl.ds(boff, BLK)], src_blk)
        pltpu.sync_copy(dst_hbm.at[pl.ds(boff, BLK)], dst_blk)

        @pl.loop(0, CPB)
        def _vec(jj):
            j = b * _i32(CPB) + jj
            for g in range(CH // 16):
                off = jj * _i32(CH) + _i32(g * 16)
                src16 = src_blk[pl.ds(off, 16)]
                dst16 = dst_blk[pl.ds(off, 16)]
                sg = plsc.load_gather(s_v, [src16])
                tg = plsc.load_gather(t_v, [dst16])
                a = sg + tg
                e = jnp.maximum(a, a * 0.01)
                ee_v[j, pl.ds(g * 16, 16)] = jnp.exp(e)
                idx2_v[j, pl.ds(g * 16, 16)] = src16 * 2 + cid
                dst2_v[j, pl.ds(g * 16, 16)] = dst16

    denlo = cid * _i32(NCH // 2)
    denhi = denlo + _i32(NCH // 2)

    @pl.loop(0, NCH)
    def _chunk(j):
        pltpu.sync_copy(zflat_hbm.at[idx2_v.at[j]], rows_v)

        @pl.loop(0, CH // 16)
        def _scale(bb):
            w16 = ee_v[j, pl.ds(bb * _i32(16), 16)]
            for i_loc in range(16):
                w = w16[i_loc]
                i = bb * _i32(16) + _i32(i_loc)
                for q in range(DH // 16):
                    sl = pl.ds(q * 16, 16)
                    rows_v[i, sl] = rows_v[i, sl] * w

        @pl.when(jnp.logical_and(j >= denlo, j < denhi))
        def _den():
            pltpu.sync_copy(ee_v.at[j], den_sh.at[dst2_v.at[j]], add=True)

        pltpu.sync_copy(rows_v, hout_sh.at[dst2_v.at[j]], add=True)

    plsc.subcore_barrier()

    pltpu.sync_copy(den_sh.at[pl.ds(rowbase, ROWS_PT)],
                    den_out.at[pl.ds(cid * _i32(N_PAD) + rowbase, ROWS_PT)])
    for q in range(ROWS_PT // CH):
        rb = rowbase + _i32(q * CH)
        pltpu.sync_copy(hout_sh.at[pl.ds(rb, CH)], hout_out.at[cid, pl.ds(rb, CH)])


def _epilogue_body(hp_ref, dp_ref, out_ref):
    d = dp_ref[0] + dp_ref[1]
    dsafe = jnp.where(d == 0.0, 1.0, d)
    hcat = jnp.concatenate([hp_ref[0], hp_ref[1]], axis=1)
    out_ref[...] = hcat * (1.0 / dsafe)


def _epilogue(hout_part, dcol):
    nblk = N_PAD // 1024
    return pl.pallas_call(
        _epilogue_body,
        grid=(nblk,),
        in_specs=[
            pl.BlockSpec((2, 1024, DH), lambda i: (0, i, 0)),
            pl.BlockSpec((2, 1024, 1), lambda i: (0, i, 0)),
        ],
        out_specs=pl.BlockSpec((1024, D), lambda i: (i, 0)),
        out_shape=jax.ShapeDtypeStruct((N_PAD, D), jnp.float32),
    )(hout_part, dcol)


def kernel(h, edge_index, W_fc, W_attn):
    src = edge_index[0].astype(jnp.int32)
    dst = edge_index[1].astype(jnp.int32)
    with _enable_x64(False):
        h_pad = jnp.pad(h, ((0, N_PAD - N_NODES), (0, 0)))
        w_at2 = jnp.stack([W_attn[0, :D], W_attn[0, D:]], axis=1)

        z, st = _prologue(h_pad, W_fc, w_at2)
        zflat = z.reshape(2 * N_PAD, DH)
        s = st[0]
        t = st[1]

        den_flat, hout_part = _sc_main(zflat, s, t, src, dst)
        dcol = den_flat.reshape(2, N_PAD, 1)

        out = _epilogue(hout_part, dcol)
        return out[:N_NODES]
